# baseline (device time: 72865 ns/iter reference)
import jax
import jax.numpy as jnp
from jax import lax
from jax.experimental import pallas as pl
from jax.experimental.pallas import tpu as pltpu

N_DEV = 16
SQ = 256
D = 1024
SKV = 4096
HQ = 8
DH = 128
W = D + DH
CH = SQ // N_DEV
SCALE = 0.08838834764831843


def kernel(x, Wq, K_ext, V_ext, Wo):
    def body(x_ref, wq_ref, k_ref, v_ref, wo_ref, out_ref,
             part_ref, slots_ref, ctx_ref, oslot_ref, islots_ref,
             bsend, brecv, dsend, drecv):
        my = lax.axis_index("i")

        barrier_sem = pltpu.get_barrier_semaphore()
        for d in range(1, N_DEV):
            peer = lax.rem(my + d, N_DEV)
            pl.semaphore_signal(barrier_sem, inc=1, device_id=(peer,),
                                device_id_type=pl.DeviceIdType.MESH)

        xq = x_ref[0].astype(jnp.bfloat16)
        wq = wq_ref[...].astype(jnp.bfloat16)
        q = lax.dot_general(xq, wq, (((1,), (0,)), ((), ())),
                            preferred_element_type=jnp.float32)
        q = q * SCALE

        for h in range(HQ):
            kh = k_ref[0, :, h * DH:(h + 1) * DH].astype(
                jnp.bfloat16).reshape(16, 256, DH)
            vh = v_ref[0, :, h * DH:(h + 1) * DH].astype(
                jnp.bfloat16).reshape(16, 256, DH)
            for qb in range(4):
                kg = kh[:, qb * 64:(qb + 1) * 64, :].reshape(1024, DH)
                vg = vh[:, qb * 64:(qb + 1) * 64, :].reshape(1024, DH)
                qg = q[qb * 64:(qb + 1) * 64,
                       h * DH:(h + 1) * DH].astype(jnp.bfloat16)
                s = lax.dot_general(qg, kg, (((1,), (1,)), ((), ())),
                                    preferred_element_type=jnp.float32)
                w = jnp.exp(s)
                lh = jnp.sum(w, axis=1, keepdims=True)
                acc = lax.dot_general(w.astype(jnp.bfloat16), vg,
                                      (((1,), (0,)), ((), ())),
                                      preferred_element_type=jnp.float32)
                rows = pl.ds(qb * 64, 64)
                part_ref[rows, h * DH:(h + 1) * DH] = acc.astype(jnp.bfloat16)
                part_ref[rows, D + h:D + h + 1] = lh.astype(jnp.bfloat16)

        pl.semaphore_wait(barrier_sem, N_DEV - 1)

        b_rdmas = []
        for d in range(1, N_DEV):
            peer = lax.rem(my + d, N_DEV)
            rdma = pltpu.make_async_remote_copy(
                src_ref=part_ref.at[pl.ds(peer * CH, CH), :],
                dst_ref=slots_ref.at[d - 1],
                send_sem=bsend.at[d - 1],
                recv_sem=brecv.at[d - 1],
                device_id=(peer,),
                device_id_type=pl.DeviceIdType.MESH,
            )
            rdma.start()
            b_rdmas.append(rdma)

        for rdma in b_rdmas:
            rdma.wait_recv()

        total = part_ref[pl.ds(my * CH, CH), :].astype(jnp.float32)
        for d in range(1, N_DEV):
            total += slots_ref[d - 1].astype(jnp.float32)
        for h in range(HQ):
            num = total[:, h * DH:(h + 1) * DH]
            den = total[:, D + h:D + h + 1]
            ctx_ref[:, h * DH:(h + 1) * DH] = (num / den).astype(jnp.bfloat16)
        res = lax.dot_general(ctx_ref[...], wo_ref[...].astype(jnp.bfloat16),
                              (((1,), (0,)), ((), ())),
                              preferred_element_type=jnp.float32)
        out_ref[0, pl.ds(my * CH, CH), :] = res
        oslot_ref[...] = res.astype(jnp.bfloat16)

        d_rdmas = []
        for d in range(1, N_DEV):
            peer = lax.rem(my + d, N_DEV)
            rdma = pltpu.make_async_remote_copy(
                src_ref=oslot_ref,
                dst_ref=islots_ref.at[d - 1],
                send_sem=dsend.at[d - 1],
                recv_sem=drecv.at[d - 1],
                device_id=(peer,),
                device_id_type=pl.DeviceIdType.MESH,
            )
            rdma.start()
            d_rdmas.append(rdma)

        for d, rdma in zip(range(1, N_DEV), d_rdmas):
            rdma.wait_recv()
            src = lax.rem(my - d + N_DEV, N_DEV)
            out_ref[0, pl.ds(src * CH, CH), :] = (
                islots_ref[d - 1].astype(jnp.float32))
        for rdma in b_rdmas:
            rdma.wait_send()
        for rdma in d_rdmas:
            rdma.wait_send()

    return pl.pallas_call(
        body,
        out_shape=jax.ShapeDtypeStruct((1, SQ, D), jnp.float32),
        in_specs=[pl.BlockSpec(memory_space=pltpu.VMEM)] * 5,
        out_specs=pl.BlockSpec(memory_space=pltpu.VMEM),
        scratch_shapes=[
            pltpu.VMEM((SQ, W), jnp.bfloat16),
            pltpu.VMEM((N_DEV - 1, CH, W), jnp.bfloat16),
            pltpu.VMEM((CH, D), jnp.bfloat16),
            pltpu.VMEM((CH, D), jnp.bfloat16),
            pltpu.VMEM((N_DEV - 1, CH, D), jnp.bfloat16),
            pltpu.SemaphoreType.DMA((N_DEV - 1,)),
            pltpu.SemaphoreType.DMA((N_DEV - 1,)),
            pltpu.SemaphoreType.DMA((N_DEV - 1,)),
            pltpu.SemaphoreType.DMA((N_DEV - 1,)),
        ],
        compiler_params=pltpu.CompilerParams(
            collective_id=0, vmem_limit_bytes=128 * 1024 * 1024),
    )(x, Wq,
      K_ext.reshape(1, SKV, HQ * DH),
      V_ext.reshape(1, SKV, HQ * DH),
      Wo)


# device time: 35345 ns/iter; 2.0615x vs baseline; 2.0615x over previous
import jax
import jax.numpy as jnp
from jax import lax
from jax.experimental import pallas as pl
from jax.experimental.pallas import tpu as pltpu

N_DEV = 16
SQ = 256
D = 1024
SKV = 4096
HQ = 8
DH = 128
W = D + DH
CH = SQ // N_DEV
SCALE = 0.08838834764831843


def kernel(x, Wq, K_ext, V_ext, Wo):
    def body(x_ref, wq_ref, k_ref, v_ref, wo_ref, out_ref,
             part_ref, slots_ref, ctx_ref, oslot_ref, islots_ref,
             kstage_ref, vstage_ref, kvsems,
             bsend, brecv, dsend, drecv):
        my = lax.axis_index("i")

        kv_dmas = []
        for h in range(HQ):
            dk = pltpu.make_async_copy(
                k_ref.at[0, :, h, :], kstage_ref.at[h], kvsems.at[0, h])
            dv = pltpu.make_async_copy(
                v_ref.at[0, :, h, :], vstage_ref.at[h], kvsems.at[1, h])
            dk.start()
            dv.start()
            kv_dmas.append((dk, dv))

        barrier_sem = pltpu.get_barrier_semaphore()
        for d in range(1, N_DEV):
            peer = lax.rem(my + d, N_DEV)
            pl.semaphore_signal(barrier_sem, inc=1, device_id=(peer,),
                                device_id_type=pl.DeviceIdType.MESH)

        xq = x_ref[0].astype(jnp.bfloat16)
        wq = wq_ref[...].astype(jnp.bfloat16)
        q = lax.dot_general(xq, wq, (((1,), (0,)), ((), ())),
                            preferred_element_type=jnp.float32)
        q = q * SCALE

        STRIPS = [(0, 256), (256, 256), (512, 256), (768, W - 768)]

        def b_send_strip(t):
            col0, wid = STRIPS[t]
            for d in range(1, N_DEV):
                peer = lax.rem(my + d, N_DEV)
                rdma = pltpu.make_async_remote_copy(
                    src_ref=part_ref.at[pl.ds(peer * CH, CH),
                                        pl.ds(col0, wid)],
                    dst_ref=slots_ref.at[d - 1, :, pl.ds(col0, wid)],
                    send_sem=bsend.at[t, d - 1],
                    recv_sem=brecv.at[t, d - 1],
                    device_id=(peer,),
                    device_id_type=pl.DeviceIdType.MESH,
                )
                rdma.start()

        for h in range(HQ):
            dk, dv = kv_dmas[h]
            dk.wait()
            dv.wait()
            kg3 = kstage_ref[h].reshape(16, 4, 64, DH).transpose(
                1, 0, 2, 3).reshape(4, 1024, DH)
            vg3 = vstage_ref[h].reshape(16, 4, 64, DH).transpose(
                1, 0, 2, 3).reshape(4, 1024, DH)
            qg3 = q[:, h * DH:(h + 1) * DH].reshape(4, 64, DH)
            s = lax.dot_general(qg3.astype(jnp.bfloat16),
                                kg3.astype(jnp.bfloat16),
                                (((2,), (2,)), ((0,), (0,))),
                                preferred_element_type=jnp.float32)
            w = jnp.exp(s)
            lh = jnp.sum(w, axis=2).reshape(SQ, 1)
            acc = lax.dot_general(w.astype(jnp.bfloat16),
                                  vg3.astype(jnp.bfloat16),
                                  (((2,), (1,)), ((0,), (0,))),
                                  preferred_element_type=jnp.float32)
            part_ref[:, h * DH:(h + 1) * DH] = (
                acc.reshape(SQ, DH).astype(jnp.bfloat16))
            part_ref[:, D + h:D + h + 1] = lh.astype(jnp.bfloat16)
            if h == 1:
                pl.semaphore_wait(barrier_sem, N_DEV - 1)
            if h % 2 == 1:
                b_send_strip(h // 2)

        def b_dummy(t, d):
            col0, wid = STRIPS[t]
            return pltpu.make_async_remote_copy(
                src_ref=slots_ref.at[d - 1, :, pl.ds(col0, wid)],
                dst_ref=slots_ref.at[d - 1, :, pl.ds(col0, wid)],
                send_sem=bsend.at[t, d - 1],
                recv_sem=brecv.at[t, d - 1],
                device_id=(my,),
                device_id_type=pl.DeviceIdType.MESH,
            )

        for t in range(4):
            for d in range(1, N_DEV):
                b_dummy(t, d).wait_recv()

        total = part_ref[pl.ds(my * CH, CH), :].astype(jnp.float32)
        for d in range(1, N_DEV):
            total += slots_ref[d - 1].astype(jnp.float32)
        for h in range(HQ):
            num = total[:, h * DH:(h + 1) * DH]
            den = total[:, D + h:D + h + 1]
            ctx_ref[:, h * DH:(h + 1) * DH] = (num / den).astype(jnp.bfloat16)
        res = lax.dot_general(ctx_ref[...], wo_ref[...].astype(jnp.bfloat16),
                              (((1,), (0,)), ((), ())),
                              preferred_element_type=jnp.float32)
        out_ref[0, pl.ds(my * CH, CH), :] = res
        oslot_ref[...] = res.astype(jnp.bfloat16)

        d_rdmas = []
        for d in range(1, N_DEV):
            peer = lax.rem(my + d, N_DEV)
            rdma = pltpu.make_async_remote_copy(
                src_ref=oslot_ref,
                dst_ref=islots_ref.at[d - 1],
                send_sem=dsend.at[d - 1],
                recv_sem=drecv.at[d - 1],
                device_id=(peer,),
                device_id_type=pl.DeviceIdType.MESH,
            )
            rdma.start()
            d_rdmas.append(rdma)

        for d, rdma in zip(range(1, N_DEV), d_rdmas):
            rdma.wait_recv()
            src = lax.rem(my - d + N_DEV, N_DEV)
            out_ref[0, pl.ds(src * CH, CH), :] = (
                islots_ref[d - 1].astype(jnp.float32))
        for t in range(4):
            for d in range(1, N_DEV):
                b_dummy(t, d).wait_send()
        for rdma in d_rdmas:
            rdma.wait_send()

    return pl.pallas_call(
        body,
        out_shape=jax.ShapeDtypeStruct((1, SQ, D), jnp.float32),
        in_specs=[
            pl.BlockSpec(memory_space=pltpu.VMEM),
            pl.BlockSpec(memory_space=pltpu.VMEM),
            pl.BlockSpec(memory_space=pltpu.MemorySpace.HBM),
            pl.BlockSpec(memory_space=pltpu.MemorySpace.HBM),
            pl.BlockSpec(memory_space=pltpu.VMEM),
        ],
        out_specs=pl.BlockSpec(memory_space=pltpu.VMEM),
        scratch_shapes=[
            pltpu.VMEM((SQ, W), jnp.bfloat16),
            pltpu.VMEM((N_DEV - 1, CH, W), jnp.bfloat16),
            pltpu.VMEM((CH, D), jnp.bfloat16),
            pltpu.VMEM((CH, D), jnp.bfloat16),
            pltpu.VMEM((N_DEV - 1, CH, D), jnp.bfloat16),
            pltpu.VMEM((HQ, SKV, DH), jnp.float32),
            pltpu.VMEM((HQ, SKV, DH), jnp.float32),
            pltpu.SemaphoreType.DMA((2, HQ)),
            pltpu.SemaphoreType.DMA((4, N_DEV - 1)),
            pltpu.SemaphoreType.DMA((4, N_DEV - 1)),
            pltpu.SemaphoreType.DMA((N_DEV - 1,)),
            pltpu.SemaphoreType.DMA((N_DEV - 1,)),
        ],
        compiler_params=pltpu.CompilerParams(
            collective_id=0, vmem_limit_bytes=128 * 1024 * 1024),
    )(x, Wq, K_ext, V_ext, Wo)
